# baseline (speedup 1.0000x reference)
import functools

import jax
import jax.numpy as jnp
from jax import lax
from jax.experimental import pallas as pl
from jax.experimental.pallas import tpu as pltpu
from jax.experimental.pallas import tpu_sc as plsc

B = 8
N = 8192
S = 1024
CIN = 128
COUT = 256


_CHUNK = 512


def _fps_body(xyzt_ref, idx_ref, nxyz_ref, dists_ref):
    C = _CHUNK
    nch = N // C
    lanecol = lax.broadcasted_iota(jnp.int32, (B, C), 1)
    lane_s = lax.broadcasted_iota(jnp.int32, (B, S), 1)
    base = lax.broadcasted_iota(jnp.int32, (B, 1), 0) * N

    dists_ref[...] = jnp.full((B, N), 1e10, jnp.float32)

    def body(i, carry):
        far, cx, cy, cz = carry
        sel = lane_s == i
        idx_ref[...] = jnp.where(sel, far + base, idx_ref[...])
        nxyz_ref[0] = jnp.where(sel, cx, nxyz_ref[0])
        nxyz_ref[1] = jnp.where(sel, cy, nxyz_ref[1])
        nxyz_ref[2] = jnp.where(sel, cz, nxyz_ref[2])

        K = 4
        st = [
            [jnp.full((B, C), -jnp.inf, jnp.float32),
             jnp.zeros((B, C), jnp.int32),
             jnp.zeros((B, C), jnp.float32),
             jnp.zeros((B, C), jnp.float32),
             jnp.zeros((B, C), jnp.float32)]
            for _ in range(K)
        ]
        for v in range(nch):
            sl = pl.ds(v * C, C)
            xs = xyzt_ref[0, :, sl]
            ys = xyzt_ref[1, :, sl]
            zs = xyzt_ref[2, :, sl]
            dx = xs - cx
            dy = ys - cy
            dz = zs - cz
            d = dz * dz + dx * dx + dy * dy
            dnew = jnp.minimum(dists_ref[:, sl], d)
            dists_ref[:, sl] = dnew
            macc, iacc, ax, ay, az = st[v % K]
            upd = dnew > macc
            st[v % K] = [jnp.where(upd, dnew, macc),
                         jnp.where(upd, lanecol + (v * C), iacc),
                         jnp.where(upd, xs, ax),
                         jnp.where(upd, ys, ay),
                         jnp.where(upd, zs, az)]
        while len(st) > 1:
            nxt = []
            for a in range(0, len(st), 2):
                ma, ia, xa, ya, za = st[a]
                mb, ib, xb, yb, zb = st[a + 1]
                bet = (ma > mb) | ((ma == mb) & (ia < ib))
                nxt.append([jnp.where(bet, ma, mb),
                            jnp.where(bet, ia, ib),
                            jnp.where(bet, xa, xb),
                            jnp.where(bet, ya, yb),
                            jnp.where(bet, za, zb)])
            st = nxt
        macc, iacc, ax, ay, az = st[0]
        maxv = jnp.max(macc, axis=1, keepdims=True)
        fnew = jnp.min(jnp.where(macc == maxv, iacc, N), axis=1,
                       keepdims=True).astype(jnp.int32)
        msel = iacc == fnew
        cxn = jnp.sum(jnp.where(msel, ax, 0.0), axis=1, keepdims=True)
        cyn = jnp.sum(jnp.where(msel, ay, 0.0), axis=1, keepdims=True)
        czn = jnp.sum(jnp.where(msel, az, 0.0), axis=1, keepdims=True)
        return fnew, cxn, cyn, czn

    far0 = jnp.zeros((B, 1), jnp.int32)
    cx0 = xyzt_ref[0, :, pl.ds(0, 1)]
    cy0 = xyzt_ref[1, :, pl.ds(0, 1)]
    cz0 = xyzt_ref[2, :, pl.ds(0, 1)]
    lax.fori_loop(0, S, body, (far0, cx0, cy0, cz0))


def _fps(xyzt):
    return pl.pallas_call(
        _fps_body,
        out_shape=(
            jax.ShapeDtypeStruct((B, S), jnp.int32),
            jax.ShapeDtypeStruct((3, B, S), jnp.float32),
        ),
        scratch_shapes=[pltpu.VMEM((B, N), jnp.float32)],
    )(xyzt)


def _sc_gather(table, idx_flat):
    info = plsc.get_sparse_core_info()
    nw = info.num_cores * info.num_subcores
    bpw = (B * S) // nw
    mesh = plsc.VectorSubcoreMesh(core_axis_name="c", subcore_axis_name="s")

    @functools.partial(
        pl.kernel,
        out_type=jax.ShapeDtypeStruct((B * S, CIN), jnp.float32),
        mesh=mesh,
        scratch_types=[
            pltpu.VMEM((bpw,), jnp.int32),
            pltpu.VMEM((bpw, CIN), jnp.float32),
            pltpu.SemaphoreType.DMA,
        ],
    )
    def k(table_hbm, idx_hbm, out_hbm, idx_v, rows_v, sem):
        wid = lax.axis_index("s") * info.num_cores + lax.axis_index("c")
        basei = wid * bpw
        pltpu.sync_copy(idx_hbm.at[pl.ds(basei, bpw)], idx_v)
        pltpu.async_copy(table_hbm.at[idx_v], rows_v, sem).wait()
        pltpu.sync_copy(rows_v, out_hbm.at[pl.ds(basei, bpw)])

    return k(table, idx_flat)


def _mix_body(g_ref, w_ref, b_ref, o_ref):
    g = g_ref[0]
    w = w_ref[...]
    o = lax.dot_general(w, g, (((1,), (1,)), ((), ())),
                        preferred_element_type=jnp.float32,
                        precision=lax.Precision.HIGHEST)
    o_ref[0] = o + b_ref[...][:, :1]


def _mix(gathered, W, b2d):
    return pl.pallas_call(
        _mix_body,
        grid=(B,),
        in_specs=[
            pl.BlockSpec((1, S, CIN), lambda i: (i, 0, 0)),
            pl.BlockSpec((COUT, CIN), lambda i: (0, 0)),
            pl.BlockSpec((COUT, 8), lambda i: (0, 0)),
        ],
        out_specs=pl.BlockSpec((1, COUT, S), lambda i: (i, 0, 0)),
        out_shape=jax.ShapeDtypeStruct((B, COUT, S), jnp.float32),
    )(gathered, W, b2d)


def kernel(xyz, x, W, b):
    xyzt = jnp.transpose(xyz, (2, 0, 1))
    idx, nxyz = _fps(xyzt)
    new_xyz = jnp.transpose(nxyz, (1, 2, 0))

    table = jnp.reshape(jnp.transpose(x, (0, 2, 1)), (B * N, CIN))
    gathered = _sc_gather(table, jnp.reshape(idx, (B * S,)))
    gathered = jnp.reshape(gathered, (B, S, CIN))

    b2d = jnp.broadcast_to(b[:, None], (COUT, 8))
    new_x = _mix(gathered, W, b2d)
    return (new_xyz, new_x)


# speedup vs baseline: 1.0283x; 1.0283x over previous
import functools

import jax
import jax.numpy as jnp
from jax import lax
from jax.experimental import pallas as pl
from jax.experimental.pallas import tpu as pltpu
from jax.experimental.pallas import tpu_sc as plsc

B = 8
N = 8192
S = 1024
CIN = 128
COUT = 256


_CHUNK = 512


def _fps_body(xyzt_ref, idx_ref, nxyz_ref, dists_ref):
    C = _CHUNK
    nch = N // C
    lanecol = lax.broadcasted_iota(jnp.int32, (B, C), 1)
    lane_s = lax.broadcasted_iota(jnp.int32, (B, S), 1)
    base = lax.broadcasted_iota(jnp.int32, (B, 1), 0) * N

    dists_ref[...] = jnp.full((B, N), 1e10, jnp.float32)

    def body(i, carry):
        far, cx, cy, cz = carry
        sel = lane_s == i
        idx_ref[...] = jnp.where(sel, far + base, idx_ref[...])
        nxyz_ref[0] = jnp.where(sel, cx, nxyz_ref[0])
        nxyz_ref[1] = jnp.where(sel, cy, nxyz_ref[1])
        nxyz_ref[2] = jnp.where(sel, cz, nxyz_ref[2])

        K = 2
        st = [
            [jnp.full((B, C), -jnp.inf, jnp.float32),
             jnp.zeros((B, C), jnp.int32),
             jnp.zeros((B, C), jnp.float32),
             jnp.zeros((B, C), jnp.float32),
             jnp.zeros((B, C), jnp.float32)]
            for _ in range(K)
        ]
        for v in range(nch):
            sl = pl.ds(v * C, C)
            xs = xyzt_ref[0, :, sl]
            ys = xyzt_ref[1, :, sl]
            zs = xyzt_ref[2, :, sl]
            dx = xs - cx
            dy = ys - cy
            dz = zs - cz
            d = dz * dz + dx * dx + dy * dy
            dnew = jnp.minimum(dists_ref[:, sl], d)
            dists_ref[:, sl] = dnew
            macc, iacc, ax, ay, az = st[v % K]
            upd = dnew > macc
            st[v % K] = [jnp.where(upd, dnew, macc),
                         jnp.where(upd, lanecol + (v * C), iacc),
                         jnp.where(upd, xs, ax),
                         jnp.where(upd, ys, ay),
                         jnp.where(upd, zs, az)]
        while len(st) > 1:
            nxt = []
            for a in range(0, len(st), 2):
                ma, ia, xa, ya, za = st[a]
                mb, ib, xb, yb, zb = st[a + 1]
                bet = (ma > mb) | ((ma == mb) & (ia < ib))
                nxt.append([jnp.where(bet, ma, mb),
                            jnp.where(bet, ia, ib),
                            jnp.where(bet, xa, xb),
                            jnp.where(bet, ya, yb),
                            jnp.where(bet, za, zb)])
            st = nxt
        macc, iacc, ax, ay, az = st[0]
        maxv = jnp.max(macc, axis=1, keepdims=True)
        fnew = jnp.min(jnp.where(macc == maxv, iacc, N), axis=1,
                       keepdims=True).astype(jnp.int32)
        msel = iacc == fnew
        cxn = jnp.sum(jnp.where(msel, ax, 0.0), axis=1, keepdims=True)
        cyn = jnp.sum(jnp.where(msel, ay, 0.0), axis=1, keepdims=True)
        czn = jnp.sum(jnp.where(msel, az, 0.0), axis=1, keepdims=True)
        return fnew, cxn, cyn, czn

    far0 = jnp.zeros((B, 1), jnp.int32)
    cx0 = xyzt_ref[0, :, pl.ds(0, 1)]
    cy0 = xyzt_ref[1, :, pl.ds(0, 1)]
    cz0 = xyzt_ref[2, :, pl.ds(0, 1)]
    lax.fori_loop(0, S, body, (far0, cx0, cy0, cz0))


def _fps(xyzt):
    return pl.pallas_call(
        _fps_body,
        out_shape=(
            jax.ShapeDtypeStruct((B, S), jnp.int32),
            jax.ShapeDtypeStruct((3, B, S), jnp.float32),
        ),
        scratch_shapes=[pltpu.VMEM((B, N), jnp.float32)],
    )(xyzt)


def _sc_gather(table, idx_flat):
    info = plsc.get_sparse_core_info()
    nw = info.num_cores * info.num_subcores
    bpw = (B * S) // nw
    mesh = plsc.VectorSubcoreMesh(core_axis_name="c", subcore_axis_name="s")

    @functools.partial(
        pl.kernel,
        out_type=jax.ShapeDtypeStruct((B * S, CIN), jnp.float32),
        mesh=mesh,
        scratch_types=[
            pltpu.VMEM((bpw,), jnp.int32),
            pltpu.VMEM((bpw, CIN), jnp.float32),
            pltpu.SemaphoreType.DMA,
        ],
    )
    def k(table_hbm, idx_hbm, out_hbm, idx_v, rows_v, sem):
        wid = lax.axis_index("s") * info.num_cores + lax.axis_index("c")
        basei = wid * bpw
        pltpu.sync_copy(idx_hbm.at[pl.ds(basei, bpw)], idx_v)
        pltpu.async_copy(table_hbm.at[idx_v], rows_v, sem).wait()
        pltpu.sync_copy(rows_v, out_hbm.at[pl.ds(basei, bpw)])

    return k(table, idx_flat)


def _mix_body(g_ref, w_ref, b_ref, o_ref):
    g = g_ref[0]
    w = w_ref[...]
    o = lax.dot_general(w, g, (((1,), (1,)), ((), ())),
                        preferred_element_type=jnp.float32,
                        precision=lax.Precision.HIGHEST)
    o_ref[0] = o + b_ref[...][:, :1]


def _mix(gathered, W, b2d):
    return pl.pallas_call(
        _mix_body,
        grid=(B,),
        in_specs=[
            pl.BlockSpec((1, S, CIN), lambda i: (i, 0, 0)),
            pl.BlockSpec((COUT, CIN), lambda i: (0, 0)),
            pl.BlockSpec((COUT, 8), lambda i: (0, 0)),
        ],
        out_specs=pl.BlockSpec((1, COUT, S), lambda i: (i, 0, 0)),
        out_shape=jax.ShapeDtypeStruct((B, COUT, S), jnp.float32),
    )(gathered, W, b2d)


def kernel(xyz, x, W, b):
    xyzt = jnp.transpose(xyz, (2, 0, 1))
    idx, nxyz = _fps(xyzt)
    new_xyz = jnp.transpose(nxyz, (1, 2, 0))

    table = jnp.reshape(jnp.transpose(x, (0, 2, 1)), (B * N, CIN))
    gathered = _sc_gather(table, jnp.reshape(idx, (B * S,)))
    gathered = jnp.reshape(gathered, (B, S, CIN))

    b2d = jnp.broadcast_to(b[:, None], (COUT, 8))
    new_x = _mix(gathered, W, b2d)
    return (new_xyz, new_x)


# speedup vs baseline: 1.0834x; 1.0536x over previous
import functools

import jax
import jax.numpy as jnp
from jax import lax
from jax.experimental import pallas as pl
from jax.experimental.pallas import tpu as pltpu
from jax.experimental.pallas import tpu_sc as plsc

B = 8
N = 8192
S = 1024
CIN = 128
COUT = 256


_CHUNK = 512


def _fps_body(xyzt_ref, idx_ref, nxyz_ref, dists_ref):
    C = _CHUNK
    nch = N // C
    lanecol = lax.broadcasted_iota(jnp.int32, (B, C), 1)
    lane_s = lax.broadcasted_iota(jnp.int32, (B, S), 1)
    base = lax.broadcasted_iota(jnp.int32, (B, 1), 0) * N

    dists_ref[...] = jnp.full((B, N), 1e10, jnp.float32)

    def body(i, carry):
        far, cx, cy, cz = carry
        sel = lane_s == i
        idx_ref[...] = jnp.where(sel, far + base, idx_ref[...])
        nxyz_ref[0] = jnp.where(sel, cx, nxyz_ref[0])
        nxyz_ref[1] = jnp.where(sel, cy, nxyz_ref[1])
        nxyz_ref[2] = jnp.where(sel, cz, nxyz_ref[2])

        macc = jnp.full((B, C), -jnp.inf, jnp.float32)
        iacc = jnp.zeros((B, C), jnp.int32)
        ax = jnp.zeros((B, C), jnp.float32)
        ay = jnp.zeros((B, C), jnp.float32)
        az = jnp.zeros((B, C), jnp.float32)
        for v in range(nch):
            sl = pl.ds(v * C, C)
            xs = xyzt_ref[0, :, sl]
            ys = xyzt_ref[1, :, sl]
            zs = xyzt_ref[2, :, sl]
            dx = xs - cx
            dy = ys - cy
            dz = zs - cz
            d = dz * dz + dx * dx + dy * dy
            dnew = jnp.minimum(dists_ref[:, sl], d)
            dists_ref[:, sl] = dnew
            upd = dnew > macc
            macc = jnp.where(upd, dnew, macc)
            iacc = jnp.where(upd, lanecol + (v * C), iacc)
            ax = jnp.where(upd, xs, ax)
            ay = jnp.where(upd, ys, ay)
            az = jnp.where(upd, zs, az)
        maxv = jnp.max(macc, axis=1, keepdims=True)
        fnew = jnp.min(jnp.where(macc == maxv, iacc, N), axis=1,
                       keepdims=True).astype(jnp.int32)
        msel = iacc == fnew
        cxn = jnp.sum(jnp.where(msel, ax, 0.0), axis=1, keepdims=True)
        cyn = jnp.sum(jnp.where(msel, ay, 0.0), axis=1, keepdims=True)
        czn = jnp.sum(jnp.where(msel, az, 0.0), axis=1, keepdims=True)
        return fnew, cxn, cyn, czn

    far0 = jnp.zeros((B, 1), jnp.int32)
    cx0 = xyzt_ref[0, :, pl.ds(0, 1)]
    cy0 = xyzt_ref[1, :, pl.ds(0, 1)]
    cz0 = xyzt_ref[2, :, pl.ds(0, 1)]
    lax.fori_loop(0, S, body, (far0, cx0, cy0, cz0))


def _fps(xyzt):
    return pl.pallas_call(
        _fps_body,
        out_shape=(
            jax.ShapeDtypeStruct((B, S), jnp.int32),
            jax.ShapeDtypeStruct((3, B, S), jnp.float32),
        ),
        scratch_shapes=[pltpu.VMEM((B, N), jnp.float32)],
    )(xyzt)


def _sc_gather(table, idx_flat):
    info = plsc.get_sparse_core_info()
    nw = info.num_cores * info.num_subcores
    bpw = (B * S) // nw
    mesh = plsc.VectorSubcoreMesh(core_axis_name="c", subcore_axis_name="s")

    @functools.partial(
        pl.kernel,
        out_type=jax.ShapeDtypeStruct((B * S, CIN), jnp.float32),
        mesh=mesh,
        scratch_types=[
            pltpu.VMEM((bpw,), jnp.int32),
            pltpu.VMEM((bpw, CIN), jnp.float32),
            pltpu.SemaphoreType.DMA,
        ],
    )
    def k(table_hbm, idx_hbm, out_hbm, idx_v, rows_v, sem):
        wid = lax.axis_index("s") * info.num_cores + lax.axis_index("c")
        basei = wid * bpw
        pltpu.sync_copy(idx_hbm.at[pl.ds(basei, bpw)], idx_v)
        pltpu.async_copy(table_hbm.at[idx_v], rows_v, sem).wait()
        pltpu.sync_copy(rows_v, out_hbm.at[pl.ds(basei, bpw)])

    return k(table, idx_flat)


def _mix_body(g_ref, w_ref, b_ref, o_ref):
    g = g_ref[0]
    w = w_ref[...]
    o = lax.dot_general(w, g, (((1,), (1,)), ((), ())),
                        preferred_element_type=jnp.float32,
                        precision=lax.Precision.HIGHEST)
    o_ref[0] = o + b_ref[...][:, :1]


def _mix(gathered, W, b2d):
    return pl.pallas_call(
        _mix_body,
        grid=(B,),
        in_specs=[
            pl.BlockSpec((1, S, CIN), lambda i: (i, 0, 0)),
            pl.BlockSpec((COUT, CIN), lambda i: (0, 0)),
            pl.BlockSpec((COUT, 8), lambda i: (0, 0)),
        ],
        out_specs=pl.BlockSpec((1, COUT, S), lambda i: (i, 0, 0)),
        out_shape=jax.ShapeDtypeStruct((B, COUT, S), jnp.float32),
    )(gathered, W, b2d)


def kernel(xyz, x, W, b):
    xyzt = jnp.transpose(xyz, (2, 0, 1))
    idx, nxyz = _fps(xyzt)
    new_xyz = jnp.transpose(nxyz, (1, 2, 0))

    table = jnp.reshape(jnp.transpose(x, (0, 2, 1)), (B * N, CIN))
    gathered = _sc_gather(table, jnp.reshape(idx, (B * S,)))
    gathered = jnp.reshape(gathered, (B, S, CIN))

    b2d = jnp.broadcast_to(b[:, None], (COUT, 8))
    new_x = _mix(gathered, W, b2d)
    return (new_xyz, new_x)
